# baseline (speedup 1.0000x reference)
import functools

import jax
import jax.numpy as jnp
from jax import lax
from jax.experimental import pallas as pl
from jax.experimental.pallas import tpu as pltpu

D = 1024
R = 16


def _row_map(r, i, idx_ref):
    return (idx_ref[i * R + r], 0, 0)


def _tc_body(idx_ref, *refs):
    out = refs[R]
    for r in range(R):
        out[r, :] = refs[r][0, 0, :]


def _make_tc_gather(n_idx):
    grid = (n_idx // R,)
    in_specs = [
        pl.BlockSpec((1, 1, D), functools.partial(_row_map, r)) for r in range(R)
    ]
    out_spec = pl.BlockSpec((R, D), lambda i, idx_ref: (i, 0))
    return pl.pallas_call(
        _tc_body,
        grid_spec=pltpu.PrefetchScalarGridSpec(
            num_scalar_prefetch=1,
            grid=grid,
            in_specs=in_specs,
            out_specs=out_spec,
        ),
        out_shape=jax.ShapeDtypeStruct((n_idx, D), jnp.float32),
    )


def kernel(position_ids, table):
    pos = position_ids.reshape(-1)
    n = pos.shape[0]
    table3 = table.reshape(table.shape[0], 1, table.shape[1])
    out = _make_tc_gather(n)(pos, *([table3] * R))
    return out.reshape(position_ids.shape + (table.shape[1],))


# speedup vs baseline: 10.5673x; 10.5673x over previous
import functools

import jax
import jax.numpy as jnp
from jax import lax
from jax.experimental import pallas as pl
from jax.experimental.pallas import tpu as pltpu
from jax.experimental.pallas import tpu_sc as plsc

D = 1024
NC, NS = 2, 16
NW = NC * NS
CHUNK = 16


def _make_gather(n_idx):
    b_per_w = n_idx // NW
    n_chunks = b_per_w // CHUNK
    n_iters = n_chunks // 4
    mesh = plsc.VectorSubcoreMesh(core_axis_name="c", subcore_axis_name="s")

    @functools.partial(
        pl.kernel,
        mesh=mesh,
        out_type=jax.ShapeDtypeStruct((n_idx, D), jnp.float32),
        scratch_types=[
            pltpu.VMEM((b_per_w,), jnp.int32),
            pltpu.VMEM((CHUNK, D), jnp.float32),
            pltpu.VMEM((CHUNK, D), jnp.float32),
            pltpu.VMEM((CHUNK, D), jnp.float32),
            pltpu.VMEM((CHUNK, D), jnp.float32),
            pltpu.SemaphoreType.DMA,
            pltpu.SemaphoreType.DMA,
            pltpu.SemaphoreType.DMA,
            pltpu.SemaphoreType.DMA,
            pltpu.SemaphoreType.DMA,
            pltpu.SemaphoreType.DMA,
            pltpu.SemaphoreType.DMA,
            pltpu.SemaphoreType.DMA,
        ],
    )
    def gather(pos_hbm, table_hbm, out_hbm, idx_v,
               r0, r1, r2, r3, gs0, gs1, gs2, gs3, ws0, ws1, ws2, ws3):
        wid = lax.axis_index("s") * NC + lax.axis_index("c")
        base = wid * b_per_w
        pltpu.sync_copy(pos_hbm.at[pl.ds(base, b_per_w)], idx_v)

        def g_src(i):
            return table_hbm.at[idx_v.at[pl.ds(i * CHUNK, CHUNK)]]

        def w_dst(i):
            return out_hbm.at[pl.ds(base + i * CHUNK, CHUNK)]

        pltpu.async_copy(g_src(0), r0, gs0)
        pltpu.async_copy(g_src(1), r1, gs1)

        def step(p, _):
            i = 4 * p

            @pl.when(p > 0)
            def _():
                pltpu.make_async_copy(r2, w_dst(i - 2), ws2).wait()
                pltpu.make_async_copy(r3, w_dst(i - 1), ws3).wait()

            pltpu.async_copy(g_src(i + 2), r2, gs2)
            pltpu.async_copy(g_src(i + 3), r3, gs3)

            pltpu.make_async_copy(g_src(i), r0, gs0).wait()
            pltpu.async_copy(r0, w_dst(i), ws0)
            pltpu.make_async_copy(g_src(i + 1), r1, gs1).wait()
            pltpu.async_copy(r1, w_dst(i + 1), ws1)

            pltpu.make_async_copy(r0, w_dst(i), ws0).wait()
            pltpu.make_async_copy(r1, w_dst(i + 1), ws1).wait()

            @pl.when(p + 1 < n_iters)
            def _():
                pltpu.async_copy(g_src(i + 4), r0, gs0)
                pltpu.async_copy(g_src(i + 5), r1, gs1)

            pltpu.make_async_copy(g_src(i + 2), r2, gs2).wait()
            pltpu.async_copy(r2, w_dst(i + 2), ws2)
            pltpu.make_async_copy(g_src(i + 3), r3, gs3).wait()
            pltpu.async_copy(r3, w_dst(i + 3), ws3)

            return 0

        lax.fori_loop(0, n_iters, step, 0)

        last = n_chunks - 2
        pltpu.make_async_copy(r2, w_dst(last), ws2).wait()
        pltpu.make_async_copy(r3, w_dst(last + 1), ws3).wait()

    return gather


def kernel(position_ids, table):
    pos = position_ids.reshape(-1)
    out = _make_gather(pos.shape[0])(pos, table)
    return out.reshape(position_ids.shape + (table.shape[1],))
